# baseline (speedup 1.0000x reference)
import functools
import math

import jax
import jax.numpy as jnp
import numpy as np
from jax.experimental import pallas as pl
from jax.experimental.pallas import tpu as pltpu

_B, _L, _H, _D = 2, 2048, 12, 64
_BH = _B * _H
_FACTOR = 5
_UPART = min(_FACTOR * int(np.ceil(np.log(_L))), _L)
_U = min(_FACTOR * int(np.ceil(np.log(_L))), _L)
_TK = 512


def _threefry2x32(k0, k1, x0, x1):
    rot = [[13, 15, 26, 6], [17, 29, 16, 24]]
    ks = [np.uint32(k0), np.uint32(k1), np.uint32(k0 ^ k1 ^ np.uint32(0x1BD11BDA))]
    x0 = (x0 + ks[0]).astype(np.uint32)
    x1 = (x1 + ks[1]).astype(np.uint32)
    for i in range(5):
        for r in rot[i % 2]:
            x0 = (x0 + x1).astype(np.uint32)
            x1 = ((x1 << np.uint32(r)) | (x1 >> np.uint32(32 - r))).astype(np.uint32)
            x1 = x1 ^ x0
        x0 = (x0 + ks[(i + 1) % 3]).astype(np.uint32)
        x1 = (x1 + ks[(i + 2) % 3] + np.uint32(i + 1)).astype(np.uint32)
    return x0, x1


def _np_randint_key42(shape, span):
    s0, s1 = _threefry2x32(np.uint32(0), np.uint32(42),
                           np.zeros(2, np.uint32), np.arange(2, dtype=np.uint32))
    lk0, lk1 = s0[1], s1[1]
    n = int(np.prod(shape))
    b0, b1 = _threefry2x32(lk0, lk1,
                           np.zeros(n, np.uint32), np.arange(n, dtype=np.uint32))
    return ((b0 ^ b1) % np.uint32(span)).astype(np.int32).reshape(shape)


_IDX = _np_randint_key42((_L, _UPART), _L)
_CNT_T_NP = np.zeros((_L, _L), np.float32)
np.add.at(_CNT_T_NP, (_IDX.ravel(), np.repeat(np.arange(_L), _UPART)), 1.0)
_MOFF_T_NP = np.where(_CNT_T_NP > 0, 0.0, -np.inf).astype(np.float32)


def _fused_body(q_ref, k_ref, v_ref, cnt_ref, moff_ref, out_ref,
                m_scr, idx_scr):
    p = pl.program_id(0)
    i = pl.program_id(1)

    @pl.when(p == 0)
    def _phase0():
        q = q_ref[0]
        run_max = jnp.full((1, _L), -jnp.inf, jnp.float32)
        run_sum = jnp.zeros((1, _L), jnp.float32)
        for t in range(_L // _TK):
            kt = k_ref[0, t * _TK:(t + 1) * _TK, :]
            s = jax.lax.dot_general(kt, q, (((1,), (1,)), ((), ())),
                                    preferred_element_type=jnp.float32)
            c = cnt_ref[t * _TK:(t + 1) * _TK, :]
            mo = moff_ref[t * _TK:(t + 1) * _TK, :]
            run_max = jnp.maximum(run_max, jnp.max(s + mo, axis=0, keepdims=True))
            run_sum = run_sum + jnp.sum(s * c, axis=0, keepdims=True)
        m_scr[pl.ds(i, 1), :] = run_max - run_sum * (1.0 / _L)

    @pl.when(jnp.logical_and(p == 0, i == _BH - 1))
    def _topk():
        m = m_scr[...]
        iota = jax.lax.broadcasted_iota(jnp.int32, (_BH, _L), 1)
        cols = []
        for _ in range(_U):
            cur = jnp.max(m, axis=1, keepdims=True)
            hit = m == cur
            pos = jnp.min(jnp.where(hit, iota, _L), axis=1, keepdims=True)
            cols.append(pos)
            m = jnp.where(iota == pos, -jnp.inf, m)
        idx_scr[...] = jnp.concatenate(cols, axis=1)

    @pl.when(p == 1)
    def _phase1():
        q = q_ref[0]
        k = k_ref[0]
        v = v_ref[0]
        rsel = jax.lax.broadcasted_iota(jnp.int32, (_BH, _U), 0) == i
        idxr = jnp.max(jnp.where(rsel, idx_scr[...], 0), axis=0, keepdims=True)
        iota_l = jax.lax.broadcasted_iota(jnp.int32, (_L, _U), 0)
        ohb = (iota_l == idxr).astype(jnp.bfloat16)
        qsel = jax.lax.dot_general(ohb, q, (((0,), (0,)), ((), ())),
                                   preferred_element_type=jnp.float32)
        scores = jax.lax.dot_general(qsel.astype(jnp.bfloat16), k,
                                     (((1,), (1,)), ((), ())),
                                     preferred_element_type=jnp.float32)
        scores = scores * (1.0 / math.sqrt(_D))
        smax = jnp.max(scores, axis=1, keepdims=True)
        e = jnp.exp(scores - smax)
        attn = e / jnp.sum(e, axis=1, keepdims=True)
        upd = jax.lax.dot_general(attn.astype(jnp.bfloat16), v,
                                  (((1,), (0,)), ((), ())),
                                  preferred_element_type=jnp.float32)
        vmean = jnp.mean(v.astype(jnp.float32), axis=0, keepdims=True)
        notsel = 1.0 - jnp.max(ohb, axis=1, keepdims=True).astype(jnp.float32)
        oh_aug = jnp.concatenate([ohb, notsel.astype(jnp.bfloat16)], axis=1)
        rows = jnp.concatenate([upd, vmean], axis=0)
        rows_hi = rows.astype(jnp.bfloat16)
        rows_lo = (rows - rows_hi.astype(jnp.float32)).astype(jnp.bfloat16)
        out_ref[0] = (
            jax.lax.dot_general(oh_aug, rows_hi, (((1,), (0,)), ((), ())),
                                preferred_element_type=jnp.float32)
            + jax.lax.dot_general(oh_aug, rows_lo, (((1,), (0,)), ((), ())),
                                  preferred_element_type=jnp.float32))


@functools.partial(jax.jit, static_argnames=())
def kernel(queries, keys, values, attn_mask):
    del attn_mask
    cnt_t = jnp.asarray(_CNT_T_NP)
    moff_t = jnp.asarray(_MOFF_T_NP)
    q_bf = jnp.transpose(queries, (0, 2, 1, 3)).reshape(_BH, _L, _D).astype(jnp.bfloat16)
    k_bf = jnp.transpose(keys, (0, 2, 1, 3)).reshape(_BH, _L, _D).astype(jnp.bfloat16)
    v_bf = jnp.transpose(values, (0, 2, 1, 3)).reshape(_BH, _L, _D).astype(jnp.bfloat16)

    qk_spec = pl.BlockSpec((1, _L, _D), lambda p, i: (i, 0, 0))
    v_spec = pl.BlockSpec((1, _L, _D), lambda p, i: (i * p, 0, 0))
    context = pl.pallas_call(
        _fused_body,
        grid=(2, _BH),
        in_specs=[qk_spec, qk_spec, v_spec,
                  pl.BlockSpec((_L, _L), lambda p, i: (0, 0)),
                  pl.BlockSpec((_L, _L), lambda p, i: (0, 0))],
        out_specs=pl.BlockSpec((1, _L, _D), lambda p, i: (i * p, 0, 0)),
        out_shape=jax.ShapeDtypeStruct((_BH, _L, _D), jnp.float32),
        scratch_shapes=[pltpu.VMEM((_BH, _L), jnp.float32),
                        pltpu.VMEM((_BH, _U), jnp.int32)],
    )(q_bf, k_bf, v_bf, cnt_t, moff_t)
    return context.reshape(_B, _H, _L, _D)
